# Initial kernel scaffold; baseline (speedup 1.0000x reference)
import jax
import jax.numpy as jnp
from jax.experimental import pallas as pl


def kernel(x, coords, W1, g1, b1, W2, g2, b2, W3, g3, b3, W4, g4, b4, Wd, bd):
    raise NotImplementedError("write your pallas kernel here")


# speedup vs baseline: 7.7443x; 7.7443x over previous
import functools

import jax
import jax.numpy as jnp
from jax import lax
from jax.experimental import pallas as pl
from jax.experimental.pallas import tpu as pltpu
from jax.experimental.pallas import tpu_sc as plsc

_NB = 9
_SEL = _NB + 1
_TN = 256
_TM = 512
_NC, _NS = 2, 16
_G = 64


def _bf(x):
    return x.astype(jnp.bfloat16)


def _knn_body(n, xa_ref, xt_ref, o_ref):
    b = pl.program_id(0)
    a = xa_ref[0]
    rt = xt_ref[0]
    inner = lax.dot_general(_bf(rt), _bf(a), (((0,), (0,)), ((), ())),
                            preferred_element_type=jnp.float32)
    sq = jnp.sum(a * a, axis=0, keepdims=True)
    sqr = jnp.sum(rt * rt, axis=0)[:, None]
    dist = (sqr + sq) - 2.0 * inner

    iota = lax.broadcasted_iota(jnp.int32, (_TN, n), 1)
    coli = lax.broadcasted_iota(jnp.int32, (_TN, 16), 1)
    cols = jnp.zeros((_TN, 16), jnp.int32)
    for t in range(_SEL):
        m = jnp.min(dist, axis=1, keepdims=True)
        am = jnp.min(jnp.where(dist == m, iota, n), axis=1, keepdims=True)
        cols = jnp.where(coli == t, am + b * n, cols)
        dist = jnp.where(iota == am, jnp.float32(jnp.inf), dist)
    o_ref[0] = cols


def _knn(x8):
    bsz, _, n = x8.shape
    return pl.pallas_call(
        functools.partial(_knn_body, n),
        grid=(bsz, n // _TN),
        in_specs=[
            pl.BlockSpec((1, 8, n), lambda b, j: (b, 0, 0)),
            pl.BlockSpec((1, 8, _TN), lambda b, j: (b, 0, j)),
        ],
        out_specs=pl.BlockSpec((1, _TN, 16), lambda b, j: (b, j, 0)),
        out_shape=jax.ShapeDtypeStruct((bsz, n, 16), jnp.int32),
    )(x8, x8)


def _sc_gather(h, idx_kflat):
    bn, c = h.shape
    nw = _NC * _NS
    ppw = bn // nw
    nchunk = ppw // _G
    mesh = plsc.VectorSubcoreMesh(core_axis_name="c", subcore_axis_name="s",
                                  num_cores=_NC, num_subcores=_NS)

    @functools.partial(
        pl.kernel,
        out_type=jax.ShapeDtypeStruct((_NB, bn, c), jnp.float32),
        mesh=mesh,
        compiler_params=pltpu.CompilerParams(use_tc_tiling_on_sc=False),
        scratch_types=[
            pltpu.VMEM((_G,), jnp.int32),
            pltpu.VMEM((_G, c), jnp.float32),
            pltpu.SemaphoreType.DMA,
        ])
    def k(h_hbm, idx_hbm, nb_hbm, idx_v, rows_v, sem):
        wid = lax.axis_index("s") * _NC + lax.axis_index("c")
        base = wid * ppw

        def chunk(ci, carry):
            pt0 = base + ci * _G
            for kk in range(_NB):
                pltpu.sync_copy(idx_hbm.at[pl.ds(kk * bn + pt0, _G)], idx_v)
                pltpu.async_copy(h_hbm.at[idx_v], rows_v, sem).wait()
                pltpu.sync_copy(rows_v, nb_hbm.at[kk, pl.ds(pt0, _G)])
            return carry

        lax.fori_loop(0, nchunk, chunk, 0)

    return k(h, idx_kflat)


def _conv_body(h_ref, nb_ref, wa_ref, wb_ref, u_ref, mx_ref, st_ref, m1_sc):
    kk = pl.program_id(1)
    h = h_ref[...]
    d = _bf(nb_ref[0] - h)
    m2 = jnp.dot(d, _bf(wb_ref[...]), preferred_element_type=jnp.float32)

    @pl.when(kk == 0)
    def _():
        m1_sc[...] = jnp.dot(_bf(h), _bf(wa_ref[...]),
                             preferred_element_type=jnp.float32)
        u_ref[...] = m1_sc[...]
        mx_ref[...] = m2

    @pl.when(kk > 0)
    def _():
        mx_ref[...] = jnp.maximum(mx_ref[...], m2)

    @pl.when((pl.program_id(0) == 0) & (kk == 0))
    def _():
        st_ref[...] = jnp.zeros_like(st_ref)

    m1 = m1_sc[...]
    t1 = jnp.sum(m2, axis=0, keepdims=True)
    t2 = jnp.sum(2.0 * m1 * m2 + m2 * m2, axis=0, keepdims=True)

    @pl.when(kk == 0)
    def _():
        st_ref[...] += jnp.concatenate(
            [jnp.sum(9.0 * m1, axis=0, keepdims=True),
             jnp.sum(9.0 * m1 * m1, axis=0, keepdims=True),
             jnp.zeros((6, m1.shape[1]), jnp.float32)], axis=0)

    st_ref[...] += jnp.concatenate(
        [t1, t2, jnp.zeros((6, t1.shape[1]), jnp.float32)], axis=0)


def _conv(h, nb, wa, wb):
    bn, cin = h.shape
    cout = wa.shape[1]
    fl = jax.ShapeDtypeStruct((bn, cout), jnp.float32)
    return pl.pallas_call(
        _conv_body,
        grid=(bn // _TM, _NB),
        in_specs=[
            pl.BlockSpec((_TM, cin), lambda i, k: (i, 0)),
            pl.BlockSpec((1, _TM, cin), lambda i, k: (k, i, 0)),
            pl.BlockSpec((cin, cout), lambda i, k: (0, 0)),
            pl.BlockSpec((cin, cout), lambda i, k: (0, 0)),
        ],
        out_specs=[
            pl.BlockSpec((_TM, cout), lambda i, k: (i, 0)),
            pl.BlockSpec((_TM, cout), lambda i, k: (i, 0)),
            pl.BlockSpec((8, cout), lambda i, k: (0, 0)),
        ],
        out_shape=[fl, fl, jax.ShapeDtypeStruct((8, cout), jnp.float32)],
        scratch_shapes=[pltpu.VMEM((_TM, cout), jnp.float32)],
    )(h, nb, wa, wb)


def _norm_body(cnt, u_ref, m_ref, st_ref, gb_ref, o_ref):
    st = st_ref[...]
    mean = st[0:1, :] / cnt
    ex2 = st[1:2, :] / cnt
    var = ex2 - mean * mean
    inv = 1.0 / jnp.sqrt(var + 1e-5)
    pre = ((u_ref[...] + m_ref[...]) - mean) * inv * gb_ref[0:1, :] + gb_ref[1:2, :]
    o_ref[...] = jnp.maximum(pre, 0.0)


def _norm(u, m, st, gb):
    bn, c = u.shape
    cnt = float(_NB * bn)
    return pl.pallas_call(
        functools.partial(_norm_body, cnt),
        grid=(bn // _TM,),
        in_specs=[
            pl.BlockSpec((_TM, c), lambda i: (i, 0)),
            pl.BlockSpec((_TM, c), lambda i: (i, 0)),
            pl.BlockSpec((8, c), lambda i: (0, 0)),
            pl.BlockSpec((8, c), lambda i: (0, 0)),
        ],
        out_specs=pl.BlockSpec((_TM, c), lambda i: (i, 0)),
        out_shape=jax.ShapeDtypeStruct((bn, c), jnp.float32),
    )(u, m, st, gb)


def _dec_body(x_ref, w_ref, bb_ref, o_ref):
    o_ref[...] = (jnp.dot(_bf(x_ref[...]), _bf(w_ref[...]),
                          preferred_element_type=jnp.float32)
                  + bb_ref[0:1, :])


def _decode(dec, wd_p, bb_p):
    bn, kin = dec.shape
    cout = wd_p.shape[1]
    return pl.pallas_call(
        _dec_body,
        grid=(bn // _TM,),
        in_specs=[
            pl.BlockSpec((_TM, kin), lambda i: (i, 0)),
            pl.BlockSpec((kin, cout), lambda i: (0, 0)),
            pl.BlockSpec((8, cout), lambda i: (0, 0)),
        ],
        out_specs=pl.BlockSpec((_TM, cout), lambda i: (i, 0)),
        out_shape=jax.ShapeDtypeStruct((bn, cout), jnp.float32),
    )(dec, wd_p, bb_p)


def _gb(gamma, beta):
    c = gamma.shape[0]
    return jnp.concatenate(
        [gamma[None, :], beta[None, :], jnp.zeros((6, c), jnp.float32)], axis=0)


def _edge_layer(h, idx_kflat, w, gamma, beta):
    cin = w.shape[1] // 2
    wa, wb = w[:, :cin].T, w[:, cin:].T
    if h.shape[1] != cin:
        pad = h.shape[1] - cin
        wa = jnp.concatenate([wa, jnp.zeros((pad, wa.shape[1]), wa.dtype)], axis=0)
        wb = jnp.concatenate([wb, jnp.zeros((pad, wb.shape[1]), wb.dtype)], axis=0)
    nb = _sc_gather(h, idx_kflat)
    u, mx, st = _conv(h, nb, wa, wb)
    return _norm(u, mx, st, _gb(gamma, beta))


def kernel(x, coords, W1, g1, b1, W2, g2, b2, W3, g3, b3, W4, g4, b4, Wd, bd):
    bsz, c0, n = x.shape
    bn = bsz * n

    x8 = jnp.concatenate([x, jnp.zeros((bsz, 8 - c0, n), x.dtype)], axis=1)
    idx16 = _knn(x8)
    idx_kflat = jnp.transpose(idx16[:, :, 1:_SEL].reshape(bn, _NB)).reshape(bn * _NB)

    h = jnp.concatenate(
        [jnp.transpose(x, (0, 2, 1)).reshape(bn, c0),
         jnp.zeros((bn, 16 - c0), x.dtype)], axis=1)

    h1 = _edge_layer(h, idx_kflat, W1, g1, b1)
    h2 = _edge_layer(h1, idx_kflat, W2, g2, b2)
    h3 = _edge_layer(h2, idx_kflat, W3, g3, b3)
    h4 = _edge_layer(h3, idx_kflat, W4, g4, b4)

    coords_t = jnp.transpose(coords, (0, 2, 1)).reshape(bn, c0)
    dec = jnp.concatenate([h4, h1, coords_t], axis=1)
    kin = dec.shape[1]
    kpad = (-kin) % 128
    dec = jnp.concatenate([dec, jnp.zeros((bn, kpad), dec.dtype)], axis=1)
    wd_p = jnp.zeros((kin + kpad, 128), jnp.float32).at[:kin, :Wd.shape[0]].set(Wd.T)
    bb_p = jnp.zeros((8, 128), jnp.float32).at[0, :Wd.shape[0]].set(bd)

    out = _decode(dec, wd_p, bb_p)[:, :Wd.shape[0]]
    return jnp.transpose(out.reshape(bsz, n, Wd.shape[0]), (0, 2, 1))


# speedup vs baseline: 9.6829x; 1.2503x over previous
import functools

import jax
import jax.numpy as jnp
from jax import lax
from jax.experimental import pallas as pl
from jax.experimental.pallas import tpu as pltpu
from jax.experimental.pallas import tpu_sc as plsc

_NB = 9
_SEL = _NB + 1
_TN = 256
_TM = 512
_NC, _NS = 2, 16
_G = 64


def _bf(x):
    return x.astype(jnp.bfloat16)


def _knn_body(n, xa_ref, xt_ref, o_ref):
    b = pl.program_id(0)
    a = xa_ref[0]
    rt = xt_ref[0]
    inner = lax.dot_general(_bf(rt), _bf(a), (((0,), (0,)), ((), ())),
                            preferred_element_type=jnp.float32)
    sq = jnp.sum(a * a, axis=0, keepdims=True)
    sqr = jnp.sum(rt * rt, axis=0)[:, None]
    dist = (sqr + sq) - 2.0 * inner

    iota = lax.broadcasted_iota(jnp.int32, (_TN, n), 1)
    coli = lax.broadcasted_iota(jnp.int32, (_TN, 16), 1)
    cols = jnp.zeros((_TN, 16), jnp.int32)
    for t in range(_SEL):
        m = jnp.min(dist, axis=1, keepdims=True)
        am = jnp.min(jnp.where(dist == m, iota, n), axis=1, keepdims=True)
        cols = jnp.where(coli == t, am + b * n, cols)
        dist = jnp.where(iota == am, jnp.float32(jnp.inf), dist)
    o_ref[0] = cols


def _knn(x8):
    bsz, _, n = x8.shape
    return pl.pallas_call(
        functools.partial(_knn_body, n),
        grid=(bsz, n // _TN),
        in_specs=[
            pl.BlockSpec((1, 8, n), lambda b, j: (b, 0, 0)),
            pl.BlockSpec((1, 8, _TN), lambda b, j: (b, 0, j)),
        ],
        out_specs=pl.BlockSpec((1, _TN, 16), lambda b, j: (b, j, 0)),
        out_shape=jax.ShapeDtypeStruct((bsz, n, 16), jnp.int32),
    )(x8, x8)


def _sc_gather(h, idx_kflat):
    bn, c = h.shape
    nw = _NC * _NS
    ppw = bn // nw
    nchunk = ppw // _G
    mesh = plsc.VectorSubcoreMesh(core_axis_name="c", subcore_axis_name="s",
                                  num_cores=_NC, num_subcores=_NS)

    @functools.partial(
        pl.kernel,
        out_type=jax.ShapeDtypeStruct((_NB, bn, c), jnp.float32),
        mesh=mesh,
        compiler_params=pltpu.CompilerParams(use_tc_tiling_on_sc=False),
        scratch_types=[
            pltpu.VMEM((_G,), jnp.int32),
            pltpu.VMEM((_G, c), jnp.float32),
            pltpu.SemaphoreType.DMA,
        ])
    def k(h_hbm, idx_hbm, nb_hbm, idx_v, rows_v, sem):
        wid = lax.axis_index("s") * _NC + lax.axis_index("c")
        base = wid * ppw

        def chunk(ci, carry):
            pt0 = base + ci * _G
            for kk in range(_NB):
                pltpu.sync_copy(idx_hbm.at[pl.ds(kk * bn + pt0, _G)], idx_v)
                pltpu.async_copy(h_hbm.at[idx_v], rows_v, sem).wait()
                pltpu.sync_copy(rows_v, nb_hbm.at[kk, pl.ds(pt0, _G)])
            return carry

        lax.fori_loop(0, nchunk, chunk, 0)

    return k(h, idx_kflat)


def _conv_body(h_ref, nb_ref, wa_ref, wb_ref, u_ref, mx_ref, st_ref, m1_sc):
    kk = pl.program_id(1)
    h = h_ref[...]
    d = _bf(nb_ref[0] - h)
    m2 = jnp.dot(d, _bf(wb_ref[...]), preferred_element_type=jnp.float32)

    @pl.when(kk == 0)
    def _():
        m1_sc[...] = jnp.dot(_bf(h), _bf(wa_ref[...]),
                             preferred_element_type=jnp.float32)
        u_ref[...] = m1_sc[...]
        mx_ref[...] = m2

    @pl.when(kk > 0)
    def _():
        mx_ref[...] = jnp.maximum(mx_ref[...], m2)

    @pl.when((pl.program_id(0) == 0) & (kk == 0))
    def _():
        st_ref[...] = jnp.zeros_like(st_ref)

    m1 = m1_sc[...]
    t1 = jnp.sum(m2, axis=0, keepdims=True)
    t2 = jnp.sum(2.0 * m1 * m2 + m2 * m2, axis=0, keepdims=True)

    @pl.when(kk == 0)
    def _():
        st_ref[...] += jnp.concatenate(
            [jnp.sum(9.0 * m1, axis=0, keepdims=True),
             jnp.sum(9.0 * m1 * m1, axis=0, keepdims=True),
             jnp.zeros((6, m1.shape[1]), jnp.float32)], axis=0)

    st_ref[...] += jnp.concatenate(
        [t1, t2, jnp.zeros((6, t1.shape[1]), jnp.float32)], axis=0)


def _conv(h, nb, wa, wb):
    bn, cin = h.shape
    cout = wa.shape[1]
    tm = 2048
    fl = jax.ShapeDtypeStruct((bn, cout), jnp.float32)
    return pl.pallas_call(
        _conv_body,
        grid=(bn // tm, _NB),
        in_specs=[
            pl.BlockSpec((tm, cin), lambda i, k: (i, 0)),
            pl.BlockSpec((1, tm, cin), lambda i, k: (k, i, 0)),
            pl.BlockSpec((cin, cout), lambda i, k: (0, 0)),
            pl.BlockSpec((cin, cout), lambda i, k: (0, 0)),
        ],
        out_specs=[
            pl.BlockSpec((tm, cout), lambda i, k: (i, 0)),
            pl.BlockSpec((tm, cout), lambda i, k: (i, 0)),
            pl.BlockSpec((8, cout), lambda i, k: (0, 0)),
        ],
        out_shape=[fl, fl, jax.ShapeDtypeStruct((8, cout), jnp.float32)],
        scratch_shapes=[pltpu.VMEM((tm, cout), jnp.float32)],
    )(h, nb, wa, wb)


def _norm_body(cnt, u_ref, m_ref, st_ref, gb_ref, o_ref):
    st = st_ref[...]
    mean = st[0:1, :] / cnt
    ex2 = st[1:2, :] / cnt
    var = ex2 - mean * mean
    inv = 1.0 / jnp.sqrt(var + 1e-5)
    pre = ((u_ref[...] + m_ref[...]) - mean) * inv * gb_ref[0:1, :] + gb_ref[1:2, :]
    o_ref[...] = jnp.maximum(pre, 0.0)


def _norm(u, m, st, gb):
    bn, c = u.shape
    cnt = float(_NB * bn)
    return pl.pallas_call(
        functools.partial(_norm_body, cnt),
        grid=(bn // _TM,),
        in_specs=[
            pl.BlockSpec((_TM, c), lambda i: (i, 0)),
            pl.BlockSpec((_TM, c), lambda i: (i, 0)),
            pl.BlockSpec((8, c), lambda i: (0, 0)),
            pl.BlockSpec((8, c), lambda i: (0, 0)),
        ],
        out_specs=pl.BlockSpec((_TM, c), lambda i: (i, 0)),
        out_shape=jax.ShapeDtypeStruct((bn, c), jnp.float32),
    )(u, m, st, gb)


def _dec_body(x_ref, w_ref, bb_ref, o_ref):
    o_ref[...] = (jnp.dot(_bf(x_ref[...]), _bf(w_ref[...]),
                          preferred_element_type=jnp.float32)
                  + bb_ref[0:1, :])


def _decode(dec, wd_p, bb_p):
    bn, kin = dec.shape
    cout = wd_p.shape[1]
    return pl.pallas_call(
        _dec_body,
        grid=(bn // _TM,),
        in_specs=[
            pl.BlockSpec((_TM, kin), lambda i: (i, 0)),
            pl.BlockSpec((kin, cout), lambda i: (0, 0)),
            pl.BlockSpec((8, cout), lambda i: (0, 0)),
        ],
        out_specs=pl.BlockSpec((_TM, cout), lambda i: (i, 0)),
        out_shape=jax.ShapeDtypeStruct((bn, cout), jnp.float32),
    )(dec, wd_p, bb_p)


def _gb(gamma, beta):
    c = gamma.shape[0]
    return jnp.concatenate(
        [gamma[None, :], beta[None, :], jnp.zeros((6, c), jnp.float32)], axis=0)


def _edge_layer(h, idx_kflat, w, gamma, beta):
    cin = w.shape[1] // 2
    wa, wb = w[:, :cin].T, w[:, cin:].T
    if h.shape[1] != cin:
        pad = h.shape[1] - cin
        wa = jnp.concatenate([wa, jnp.zeros((pad, wa.shape[1]), wa.dtype)], axis=0)
        wb = jnp.concatenate([wb, jnp.zeros((pad, wb.shape[1]), wb.dtype)], axis=0)
    nb = _sc_gather(h, idx_kflat)
    u, mx, st = _conv(h, nb, wa, wb)
    return _norm(u, mx, st, _gb(gamma, beta))


def kernel(x, coords, W1, g1, b1, W2, g2, b2, W3, g3, b3, W4, g4, b4, Wd, bd):
    bsz, c0, n = x.shape
    bn = bsz * n

    x8 = jnp.concatenate([x, jnp.zeros((bsz, 8 - c0, n), x.dtype)], axis=1)
    idx16 = _knn(x8)
    idx_kflat = jnp.transpose(idx16[:, :, 1:_SEL].reshape(bn, _NB)).reshape(bn * _NB)

    h = jnp.concatenate(
        [jnp.transpose(x, (0, 2, 1)).reshape(bn, c0),
         jnp.zeros((bn, 16 - c0), x.dtype)], axis=1)

    h1 = _edge_layer(h, idx_kflat, W1, g1, b1)
    h2 = _edge_layer(h1, idx_kflat, W2, g2, b2)
    h3 = _edge_layer(h2, idx_kflat, W3, g3, b3)
    h4 = _edge_layer(h3, idx_kflat, W4, g4, b4)

    coords_t = jnp.transpose(coords, (0, 2, 1)).reshape(bn, c0)
    dec = jnp.concatenate([h4, h1, coords_t], axis=1)
    kin = dec.shape[1]
    kpad = (-kin) % 128
    dec = jnp.concatenate([dec, jnp.zeros((bn, kpad), dec.dtype)], axis=1)
    wd_p = jnp.zeros((kin + kpad, 128), jnp.float32).at[:kin, :Wd.shape[0]].set(Wd.T)
    bb_p = jnp.zeros((8, 128), jnp.float32).at[0, :Wd.shape[0]].set(bd)

    out = _decode(dec, wd_p, bb_p)[:, :Wd.shape[0]]
    return jnp.transpose(out.reshape(bsz, n, Wd.shape[0]), (0, 2, 1))


# speedup vs baseline: 11.4519x; 1.1827x over previous
import functools

import jax
import jax.numpy as jnp
from jax import lax
from jax.experimental import pallas as pl
from jax.experimental.pallas import tpu as pltpu
from jax.experimental.pallas import tpu_sc as plsc

_NB = 9
_SEL = _NB + 1
_TN = 256
_TM = 512
_NC, _NS = 2, 16
_G = 64


def _bf(x):
    return x.astype(jnp.bfloat16)


def _knn_body(n, xa_ref, xt_ref, o_ref):
    b = pl.program_id(0)
    a = xa_ref[0]
    rt = xt_ref[0]
    inner = lax.dot_general(_bf(rt), _bf(a), (((0,), (0,)), ((), ())),
                            preferred_element_type=jnp.float32)
    sq = jnp.sum(a * a, axis=0, keepdims=True)
    sqr = jnp.sum(rt * rt, axis=0)[:, None]
    dist = (sqr + sq) - 2.0 * inner

    iota = lax.broadcasted_iota(jnp.int32, (_TN, n), 1)
    coli = lax.broadcasted_iota(jnp.int32, (_TN, 16), 1)
    cols = jnp.zeros((_TN, 16), jnp.int32)
    for t in range(_SEL):
        m = jnp.min(dist, axis=1, keepdims=True)
        am = jnp.min(jnp.where(dist == m, iota, n), axis=1, keepdims=True)
        cols = jnp.where(coli == t, am + b * n, cols)
        dist = jnp.where(iota == am, jnp.float32(jnp.inf), dist)
    o_ref[0] = cols


def _knn(x8):
    bsz, _, n = x8.shape
    return pl.pallas_call(
        functools.partial(_knn_body, n),
        grid=(bsz, n // _TN),
        in_specs=[
            pl.BlockSpec((1, 8, n), lambda b, j: (b, 0, 0)),
            pl.BlockSpec((1, 8, _TN), lambda b, j: (b, 0, j)),
        ],
        out_specs=pl.BlockSpec((1, _TN, 16), lambda b, j: (b, j, 0)),
        out_shape=jax.ShapeDtypeStruct((bsz, n, 16), jnp.int32),
    )(x8, x8)


def _sc_gather(h, idx_kflat):
    bn, c = h.shape
    nw = _NC * _NS
    ppw = bn // nw
    nchunk = ppw // _G
    mesh = plsc.VectorSubcoreMesh(core_axis_name="c", subcore_axis_name="s",
                                  num_cores=_NC, num_subcores=_NS)

    gg = 128
    nch = ppw // gg
    t_total = _NB * nch

    @functools.partial(
        pl.kernel,
        out_type=jax.ShapeDtypeStruct((_NB, bn, c), jnp.float32),
        mesh=mesh,
        compiler_params=pltpu.CompilerParams(use_tc_tiling_on_sc=False),
        scratch_types=[
            pltpu.VMEM((_NB * ppw,), jnp.int32),
            pltpu.VMEM((gg, c), jnp.float32),
            pltpu.VMEM((gg, c), jnp.float32),
            pltpu.SemaphoreType.DMA,
            pltpu.SemaphoreType.DMA,
            pltpu.SemaphoreType.DMA,
            pltpu.SemaphoreType.DMA,
        ])
    def k(h_hbm, idx_hbm, nb_hbm, idx_all, rows0, rows1, g0, g1, s0, s1):
        wid = lax.axis_index("s") * _NC + lax.axis_index("c")
        base = wid * ppw
        rows = (rows0, rows1)
        gsem = (g0, g1)
        ssem = (s0, s1)

        for kk in range(_NB):
            pltpu.sync_copy(idx_hbm.at[pl.ds(kk * bn + base, ppw)],
                            idx_all.at[pl.ds(kk * ppw, ppw)])

        gath = [None, None]
        stor = [None, None]
        for t in range(t_total):
            b = t % 2
            if t >= 2:
                stor[b].wait()
            kk, cc = divmod(t, nch)
            gath[b] = pltpu.async_copy(
                h_hbm.at[idx_all.at[pl.ds(kk * ppw + cc * gg, gg)]],
                rows[b], gsem[b])
            if t >= 1:
                bp = (t - 1) % 2
                gath[bp].wait()
                kp, cp = divmod(t - 1, nch)
                stor[bp] = pltpu.async_copy(
                    rows[bp], nb_hbm.at[kp, pl.ds(base + cp * gg, gg)],
                    ssem[bp])
        bl = (t_total - 1) % 2
        gath[bl].wait()
        kp, cp = divmod(t_total - 1, nch)
        stor[bl] = pltpu.async_copy(
            rows[bl], nb_hbm.at[kp, pl.ds(base + cp * gg, gg)], ssem[bl])
        stor[1 - bl].wait()
        stor[bl].wait()

    return k(h, idx_kflat)


def _conv_body(h_ref, nb_ref, wa_ref, wb_ref, u_ref, mx_ref, st_ref, m1_sc):
    kk = pl.program_id(1)
    h = h_ref[...]
    d = _bf(nb_ref[0] - h)
    m2 = jnp.dot(d, _bf(wb_ref[...]), preferred_element_type=jnp.float32)

    @pl.when(kk == 0)
    def _():
        m1_sc[...] = jnp.dot(_bf(h), _bf(wa_ref[...]),
                             preferred_element_type=jnp.float32)
        u_ref[...] = m1_sc[...]
        mx_ref[...] = m2

    @pl.when(kk > 0)
    def _():
        mx_ref[...] = jnp.maximum(mx_ref[...], m2)

    @pl.when((pl.program_id(0) == 0) & (kk == 0))
    def _():
        st_ref[...] = jnp.zeros_like(st_ref)

    m1 = m1_sc[...]
    t1 = jnp.sum(m2, axis=0, keepdims=True)
    t2 = jnp.sum(2.0 * m1 * m2 + m2 * m2, axis=0, keepdims=True)

    @pl.when(kk == 0)
    def _():
        st_ref[...] += jnp.concatenate(
            [jnp.sum(9.0 * m1, axis=0, keepdims=True),
             jnp.sum(9.0 * m1 * m1, axis=0, keepdims=True),
             jnp.zeros((6, m1.shape[1]), jnp.float32)], axis=0)

    st_ref[...] += jnp.concatenate(
        [t1, t2, jnp.zeros((6, t1.shape[1]), jnp.float32)], axis=0)


def _conv(h, nb, wa, wb):
    bn, cin = h.shape
    cout = wa.shape[1]
    tm = 2048
    fl = jax.ShapeDtypeStruct((bn, cout), jnp.float32)
    return pl.pallas_call(
        _conv_body,
        grid=(bn // tm, _NB),
        in_specs=[
            pl.BlockSpec((tm, cin), lambda i, k: (i, 0)),
            pl.BlockSpec((1, tm, cin), lambda i, k: (k, i, 0)),
            pl.BlockSpec((cin, cout), lambda i, k: (0, 0)),
            pl.BlockSpec((cin, cout), lambda i, k: (0, 0)),
        ],
        out_specs=[
            pl.BlockSpec((tm, cout), lambda i, k: (i, 0)),
            pl.BlockSpec((tm, cout), lambda i, k: (i, 0)),
            pl.BlockSpec((8, cout), lambda i, k: (0, 0)),
        ],
        out_shape=[fl, fl, jax.ShapeDtypeStruct((8, cout), jnp.float32)],
        scratch_shapes=[pltpu.VMEM((tm, cout), jnp.float32)],
    )(h, nb, wa, wb)


def _norm_body(cnt, u_ref, m_ref, st_ref, gb_ref, o_ref):
    st = st_ref[...]
    mean = st[0:1, :] / cnt
    ex2 = st[1:2, :] / cnt
    var = ex2 - mean * mean
    inv = 1.0 / jnp.sqrt(var + 1e-5)
    pre = ((u_ref[...] + m_ref[...]) - mean) * inv * gb_ref[0:1, :] + gb_ref[1:2, :]
    o_ref[...] = jnp.maximum(pre, 0.0)


def _norm(u, m, st, gb):
    bn, c = u.shape
    cnt = float(_NB * bn)
    return pl.pallas_call(
        functools.partial(_norm_body, cnt),
        grid=(bn // _TM,),
        in_specs=[
            pl.BlockSpec((_TM, c), lambda i: (i, 0)),
            pl.BlockSpec((_TM, c), lambda i: (i, 0)),
            pl.BlockSpec((8, c), lambda i: (0, 0)),
            pl.BlockSpec((8, c), lambda i: (0, 0)),
        ],
        out_specs=pl.BlockSpec((_TM, c), lambda i: (i, 0)),
        out_shape=jax.ShapeDtypeStruct((bn, c), jnp.float32),
    )(u, m, st, gb)


def _dec_body(x_ref, w_ref, bb_ref, o_ref):
    o_ref[...] = (jnp.dot(_bf(x_ref[...]), _bf(w_ref[...]),
                          preferred_element_type=jnp.float32)
                  + bb_ref[0:1, :])


def _decode(dec, wd_p, bb_p):
    bn, kin = dec.shape
    cout = wd_p.shape[1]
    return pl.pallas_call(
        _dec_body,
        grid=(bn // _TM,),
        in_specs=[
            pl.BlockSpec((_TM, kin), lambda i: (i, 0)),
            pl.BlockSpec((kin, cout), lambda i: (0, 0)),
            pl.BlockSpec((8, cout), lambda i: (0, 0)),
        ],
        out_specs=pl.BlockSpec((_TM, cout), lambda i: (i, 0)),
        out_shape=jax.ShapeDtypeStruct((bn, cout), jnp.float32),
    )(dec, wd_p, bb_p)


def _gb(gamma, beta):
    c = gamma.shape[0]
    return jnp.concatenate(
        [gamma[None, :], beta[None, :], jnp.zeros((6, c), jnp.float32)], axis=0)


def _edge_layer(h, idx_kflat, w, gamma, beta):
    cin = w.shape[1] // 2
    wa, wb = w[:, :cin].T, w[:, cin:].T
    if h.shape[1] != cin:
        pad = h.shape[1] - cin
        wa = jnp.concatenate([wa, jnp.zeros((pad, wa.shape[1]), wa.dtype)], axis=0)
        wb = jnp.concatenate([wb, jnp.zeros((pad, wb.shape[1]), wb.dtype)], axis=0)
    nb = _sc_gather(h, idx_kflat)
    u, mx, st = _conv(h, nb, wa, wb)
    return _norm(u, mx, st, _gb(gamma, beta))


def kernel(x, coords, W1, g1, b1, W2, g2, b2, W3, g3, b3, W4, g4, b4, Wd, bd):
    bsz, c0, n = x.shape
    bn = bsz * n

    x8 = jnp.concatenate([x, jnp.zeros((bsz, 8 - c0, n), x.dtype)], axis=1)
    idx16 = _knn(x8)
    idx_kflat = jnp.transpose(idx16[:, :, 1:_SEL].reshape(bn, _NB)).reshape(bn * _NB)

    h = jnp.concatenate(
        [jnp.transpose(x, (0, 2, 1)).reshape(bn, c0),
         jnp.zeros((bn, 16 - c0), x.dtype)], axis=1)

    h1 = _edge_layer(h, idx_kflat, W1, g1, b1)
    h2 = _edge_layer(h1, idx_kflat, W2, g2, b2)
    h3 = _edge_layer(h2, idx_kflat, W3, g3, b3)
    h4 = _edge_layer(h3, idx_kflat, W4, g4, b4)

    coords_t = jnp.transpose(coords, (0, 2, 1)).reshape(bn, c0)
    dec = jnp.concatenate([h4, h1, coords_t], axis=1)
    kin = dec.shape[1]
    kpad = (-kin) % 128
    dec = jnp.concatenate([dec, jnp.zeros((bn, kpad), dec.dtype)], axis=1)
    wd_p = jnp.zeros((kin + kpad, 128), jnp.float32).at[:kin, :Wd.shape[0]].set(Wd.T)
    bb_p = jnp.zeros((8, 128), jnp.float32).at[0, :Wd.shape[0]].set(bd)

    out = _decode(dec, wd_p, bb_p)[:, :Wd.shape[0]]
    return jnp.transpose(out.reshape(bsz, n, Wd.shape[0]), (0, 2, 1))


# speedup vs baseline: 12.0046x; 1.0483x over previous
import functools

import jax
import jax.numpy as jnp
from jax import lax
from jax.experimental import pallas as pl
from jax.experimental.pallas import tpu as pltpu
from jax.experimental.pallas import tpu_sc as plsc

_NB = 9
_SEL = _NB + 1
_TN = 256
_TM = 512
_NC, _NS = 2, 16
_G = 64


def _bf(x):
    return x.astype(jnp.bfloat16)


def _knn_body(n, xa_ref, xt_ref, o_ref):
    b = pl.program_id(0)
    a = xa_ref[0]
    rt = xt_ref[0]
    inner = lax.dot_general(_bf(rt), _bf(a), (((0,), (0,)), ((), ())),
                            preferred_element_type=jnp.float32)
    sq = jnp.sum(a * a, axis=0, keepdims=True)
    sqr = jnp.sum(rt * rt, axis=0)[:, None]
    dist = (sqr + sq) - 2.0 * inner

    iota = lax.broadcasted_iota(jnp.int32, (_TN, n), 1)
    coli = lax.broadcasted_iota(jnp.int32, (_TN, 16), 1)
    cols = jnp.zeros((_TN, 16), jnp.int32)
    for t in range(_SEL):
        m = jnp.min(dist, axis=1, keepdims=True)
        am = jnp.min(jnp.where(dist == m, iota, n), axis=1, keepdims=True)
        cols = jnp.where(coli == t, am + b * n, cols)
        dist = jnp.where(iota == am, jnp.float32(jnp.inf), dist)
    o_ref[0] = cols


def _knn(x8):
    bsz, _, n = x8.shape
    return pl.pallas_call(
        functools.partial(_knn_body, n),
        grid=(bsz, n // _TN),
        in_specs=[
            pl.BlockSpec((1, 8, n), lambda b, j: (b, 0, 0)),
            pl.BlockSpec((1, 8, _TN), lambda b, j: (b, 0, j)),
        ],
        out_specs=pl.BlockSpec((1, _TN, 16), lambda b, j: (b, j, 0)),
        out_shape=jax.ShapeDtypeStruct((bsz, n, 16), jnp.int32),
    )(x8, x8)


def _sc_gather(h, idx_kflat):
    bn, c = h.shape
    nw = _NC * _NS
    ppw = bn // nw
    nchunk = ppw // _G
    mesh = plsc.VectorSubcoreMesh(core_axis_name="c", subcore_axis_name="s",
                                  num_cores=_NC, num_subcores=_NS)

    gg = 128
    nch = ppw // gg
    t_total = _NB * nch

    @functools.partial(
        pl.kernel,
        out_type=jax.ShapeDtypeStruct((_NB, bn, c), jnp.float32),
        mesh=mesh,
        compiler_params=pltpu.CompilerParams(use_tc_tiling_on_sc=False),
        scratch_types=[
            pltpu.VMEM((_NB * ppw,), jnp.int32),
            pltpu.VMEM((gg, c), jnp.float32),
            pltpu.VMEM((gg, c), jnp.float32),
            pltpu.SemaphoreType.DMA,
            pltpu.SemaphoreType.DMA,
            pltpu.SemaphoreType.DMA,
            pltpu.SemaphoreType.DMA,
        ])
    def k(h_hbm, idx_hbm, nb_hbm, idx_all, rows0, rows1, g0, g1, s0, s1):
        wid = lax.axis_index("s") * _NC + lax.axis_index("c")
        base = wid * ppw
        rows = (rows0, rows1)
        gsem = (g0, g1)
        ssem = (s0, s1)

        for kk in range(_NB):
            pltpu.sync_copy(idx_hbm.at[pl.ds(kk * bn + base, ppw)],
                            idx_all.at[pl.ds(kk * ppw, ppw)])

        gath = [None, None]
        stor = [None, None]
        for t in range(t_total):
            b = t % 2
            if t >= 2:
                stor[b].wait()
            kk, cc = divmod(t, nch)
            gath[b] = pltpu.async_copy(
                h_hbm.at[idx_all.at[pl.ds(kk * ppw + cc * gg, gg)]],
                rows[b], gsem[b])
            if t >= 1:
                bp = (t - 1) % 2
                gath[bp].wait()
                kp, cp = divmod(t - 1, nch)
                stor[bp] = pltpu.async_copy(
                    rows[bp], nb_hbm.at[kp, pl.ds(base + cp * gg, gg)],
                    ssem[bp])
        bl = (t_total - 1) % 2
        gath[bl].wait()
        kp, cp = divmod(t_total - 1, nch)
        stor[bl] = pltpu.async_copy(
            rows[bl], nb_hbm.at[kp, pl.ds(base + cp * gg, gg)], ssem[bl])
        stor[1 - bl].wait()
        stor[bl].wait()

    return k(h, idx_kflat)


def _conv_body(h_ref, nb_ref, wa_ref, wb_ref, mx_ref, st_ref, m1_sc):
    kk = pl.program_id(1)
    h = h_ref[...]

    @pl.when(kk == 0)
    def _():
        m1_sc[...] = jnp.dot(_bf(h), _bf(wa_ref[...]),
                             preferred_element_type=jnp.float32)

    d = _bf(nb_ref[0] - h)
    e = jnp.dot(d, _bf(wb_ref[...]),
                preferred_element_type=jnp.float32) + m1_sc[...]

    @pl.when(kk == 0)
    def _():
        mx_ref[...] = e

    @pl.when(kk > 0)
    def _():
        mx_ref[...] = jnp.maximum(mx_ref[...], e)

    @pl.when((pl.program_id(0) == 0) & (kk == 0))
    def _():
        st_ref[...] = jnp.zeros_like(st_ref)

    st_ref[...] += jnp.concatenate(
        [jnp.sum(e, axis=0, keepdims=True),
         jnp.sum(e * e, axis=0, keepdims=True),
         jnp.zeros((6, e.shape[1]), jnp.float32)], axis=0)


def _conv(h, nb, wa, wb):
    bn, cin = h.shape
    cout = wa.shape[1]
    tm = 2048
    fl = jax.ShapeDtypeStruct((bn, cout), jnp.float32)
    return pl.pallas_call(
        _conv_body,
        grid=(bn // tm, _NB),
        in_specs=[
            pl.BlockSpec((tm, cin), lambda i, k: (i, 0)),
            pl.BlockSpec((1, tm, cin), lambda i, k: (k, i, 0)),
            pl.BlockSpec((cin, cout), lambda i, k: (0, 0)),
            pl.BlockSpec((cin, cout), lambda i, k: (0, 0)),
        ],
        out_specs=[
            pl.BlockSpec((tm, cout), lambda i, k: (i, 0)),
            pl.BlockSpec((8, cout), lambda i, k: (0, 0)),
        ],
        out_shape=[fl, jax.ShapeDtypeStruct((8, cout), jnp.float32)],
        scratch_shapes=[pltpu.VMEM((tm, cout), jnp.float32)],
    )(h, nb, wa, wb)


def _norm_body(cnt, m_ref, st_ref, gb_ref, o_ref):
    st = st_ref[...]
    mean = st[0:1, :] / cnt
    ex2 = st[1:2, :] / cnt
    var = ex2 - mean * mean
    inv = 1.0 / jnp.sqrt(var + 1e-5)
    pre = (m_ref[...] - mean) * inv * gb_ref[0:1, :] + gb_ref[1:2, :]
    o_ref[...] = jnp.maximum(pre, 0.0)


def _norm(m, st, gb):
    bn, c = m.shape
    cnt = float(_NB * bn)
    return pl.pallas_call(
        functools.partial(_norm_body, cnt),
        grid=(bn // _TM,),
        in_specs=[
            pl.BlockSpec((_TM, c), lambda i: (i, 0)),
            pl.BlockSpec((8, c), lambda i: (0, 0)),
            pl.BlockSpec((8, c), lambda i: (0, 0)),
        ],
        out_specs=pl.BlockSpec((_TM, c), lambda i: (i, 0)),
        out_shape=jax.ShapeDtypeStruct((bn, c), jnp.float32),
    )(m, st, gb)


def _dec_body(x_ref, w_ref, bb_ref, o_ref):
    o_ref[...] = (jnp.dot(_bf(x_ref[...]), _bf(w_ref[...]),
                          preferred_element_type=jnp.float32)
                  + bb_ref[0:1, :])


def _decode(dec, wd_p, bb_p):
    bn, kin = dec.shape
    cout = wd_p.shape[1]
    return pl.pallas_call(
        _dec_body,
        grid=(bn // _TM,),
        in_specs=[
            pl.BlockSpec((_TM, kin), lambda i: (i, 0)),
            pl.BlockSpec((kin, cout), lambda i: (0, 0)),
            pl.BlockSpec((8, cout), lambda i: (0, 0)),
        ],
        out_specs=pl.BlockSpec((_TM, cout), lambda i: (i, 0)),
        out_shape=jax.ShapeDtypeStruct((bn, cout), jnp.float32),
    )(dec, wd_p, bb_p)


def _gb(gamma, beta):
    c = gamma.shape[0]
    return jnp.concatenate(
        [gamma[None, :], beta[None, :], jnp.zeros((6, c), jnp.float32)], axis=0)


def _edge_layer(h, idx_kflat, w, gamma, beta):
    cin = w.shape[1] // 2
    wa, wb = w[:, :cin].T, w[:, cin:].T
    if h.shape[1] != cin:
        pad = h.shape[1] - cin
        wa = jnp.concatenate([wa, jnp.zeros((pad, wa.shape[1]), wa.dtype)], axis=0)
        wb = jnp.concatenate([wb, jnp.zeros((pad, wb.shape[1]), wb.dtype)], axis=0)
    nb = _sc_gather(h, idx_kflat)
    mx, st = _conv(h, nb, wa, wb)
    return _norm(mx, st, _gb(gamma, beta))


def kernel(x, coords, W1, g1, b1, W2, g2, b2, W3, g3, b3, W4, g4, b4, Wd, bd):
    bsz, c0, n = x.shape
    bn = bsz * n

    x8 = jnp.concatenate([x, jnp.zeros((bsz, 8 - c0, n), x.dtype)], axis=1)
    idx16 = _knn(x8)
    idx_kflat = jnp.transpose(idx16[:, :, 1:_SEL].reshape(bn, _NB)).reshape(bn * _NB)

    h = jnp.concatenate(
        [jnp.transpose(x, (0, 2, 1)).reshape(bn, c0),
         jnp.zeros((bn, 16 - c0), x.dtype)], axis=1)

    h1 = _edge_layer(h, idx_kflat, W1, g1, b1)
    h2 = _edge_layer(h1, idx_kflat, W2, g2, b2)
    h3 = _edge_layer(h2, idx_kflat, W3, g3, b3)
    h4 = _edge_layer(h3, idx_kflat, W4, g4, b4)

    coords_t = jnp.transpose(coords, (0, 2, 1)).reshape(bn, c0)
    dec = jnp.concatenate([h4, h1, coords_t], axis=1)
    kin = dec.shape[1]
    kpad = (-kin) % 128
    dec = jnp.concatenate([dec, jnp.zeros((bn, kpad), dec.dtype)], axis=1)
    wd_p = jnp.zeros((kin + kpad, 128), jnp.float32).at[:kin, :Wd.shape[0]].set(Wd.T)
    bb_p = jnp.zeros((8, 128), jnp.float32).at[0, :Wd.shape[0]].set(bd)

    out = _decode(dec, wd_p, bb_p)[:, :Wd.shape[0]]
    return jnp.transpose(out.reshape(bsz, n, Wd.shape[0]), (0, 2, 1))


# speedup vs baseline: 12.5854x; 1.0484x over previous
import functools

import jax
import jax.numpy as jnp
from jax import lax
from jax.experimental import pallas as pl
from jax.experimental.pallas import tpu as pltpu
from jax.experimental.pallas import tpu_sc as plsc

_NB = 9
_SEL = _NB + 1
_TN = 512
_TM = 512
_NC, _NS = 2, 16
_G = 64


def _bf(x):
    return x.astype(jnp.bfloat16)


def _knn_body(n, xa_ref, xt_ref, o_ref):
    b = pl.program_id(0)
    a = xa_ref[0]
    rt = xt_ref[0]
    inner = lax.dot_general(_bf(rt), _bf(a), (((0,), (0,)), ((), ())),
                            preferred_element_type=jnp.float32)
    sq = jnp.sum(a * a, axis=0, keepdims=True)
    sqr = jnp.sum(rt * rt, axis=0)[:, None]
    dist = (sqr + sq) - 2.0 * inner

    iota = lax.broadcasted_iota(jnp.int32, (_TN, n), 1)
    coli = lax.broadcasted_iota(jnp.int32, (_TN, 16), 1)
    cols = jnp.zeros((_TN, 16), jnp.int32)
    for t in range(_SEL):
        m = jnp.min(dist, axis=1, keepdims=True)
        am = jnp.min(jnp.where(dist == m, iota, n), axis=1, keepdims=True)
        cols = jnp.where(coli == t, am + b * n, cols)
        dist = jnp.where(iota == am, jnp.float32(jnp.inf), dist)
    o_ref[0] = cols


def _knn(x8):
    bsz, _, n = x8.shape
    return pl.pallas_call(
        functools.partial(_knn_body, n),
        grid=(bsz, n // _TN),
        in_specs=[
            pl.BlockSpec((1, 8, n), lambda b, j: (b, 0, 0)),
            pl.BlockSpec((1, 8, _TN), lambda b, j: (b, 0, j)),
        ],
        out_specs=pl.BlockSpec((1, _TN, 16), lambda b, j: (b, j, 0)),
        out_shape=jax.ShapeDtypeStruct((bsz, n, 16), jnp.int32),
    )(x8, x8)


def _sc_gather(h, idx_kflat):
    bn, c = h.shape
    nw = _NC * _NS
    ppw = bn // nw
    nchunk = ppw // _G
    mesh = plsc.VectorSubcoreMesh(core_axis_name="c", subcore_axis_name="s",
                                  num_cores=_NC, num_subcores=_NS)

    gg = 128
    nch = ppw // gg
    t_total = _NB * nch

    @functools.partial(
        pl.kernel,
        out_type=jax.ShapeDtypeStruct((_NB, bn, c), jnp.float32),
        mesh=mesh,
        compiler_params=pltpu.CompilerParams(use_tc_tiling_on_sc=False),
        scratch_types=[
            pltpu.VMEM((_NB * ppw,), jnp.int32),
            pltpu.VMEM((gg, c), jnp.float32),
            pltpu.VMEM((gg, c), jnp.float32),
            pltpu.SemaphoreType.DMA,
            pltpu.SemaphoreType.DMA,
            pltpu.SemaphoreType.DMA,
            pltpu.SemaphoreType.DMA,
        ])
    def k(h_hbm, idx_hbm, nb_hbm, idx_all, rows0, rows1, g0, g1, s0, s1):
        wid = lax.axis_index("s") * _NC + lax.axis_index("c")
        base = wid * ppw
        rows = (rows0, rows1)
        gsem = (g0, g1)
        ssem = (s0, s1)

        for kk in range(_NB):
            pltpu.sync_copy(idx_hbm.at[pl.ds(kk * bn + base, ppw)],
                            idx_all.at[pl.ds(kk * ppw, ppw)])

        gath = [None, None]
        stor = [None, None]
        for t in range(t_total):
            b = t % 2
            if t >= 2:
                stor[b].wait()
            kk, cc = divmod(t, nch)
            gath[b] = pltpu.async_copy(
                h_hbm.at[idx_all.at[pl.ds(kk * ppw + cc * gg, gg)]],
                rows[b], gsem[b])
            if t >= 1:
                bp = (t - 1) % 2
                gath[bp].wait()
                kp, cp = divmod(t - 1, nch)
                stor[bp] = pltpu.async_copy(
                    rows[bp], nb_hbm.at[kp, pl.ds(base + cp * gg, gg)],
                    ssem[bp])
        bl = (t_total - 1) % 2
        gath[bl].wait()
        kp, cp = divmod(t_total - 1, nch)
        stor[bl] = pltpu.async_copy(
            rows[bl], nb_hbm.at[kp, pl.ds(base + cp * gg, gg)], ssem[bl])
        stor[1 - bl].wait()
        stor[bl].wait()

    return k(h, idx_kflat)


def _conv_body(h_ref, nb_ref, wa_ref, wb_ref, mx_ref, st_ref, m1_sc):
    kk = pl.program_id(1)
    h = h_ref[...]

    @pl.when(kk == 0)
    def _():
        m1_sc[...] = jnp.dot(_bf(h), _bf(wa_ref[...]),
                             preferred_element_type=jnp.float32)

    d = _bf(nb_ref[0] - h)
    e = jnp.dot(d, _bf(wb_ref[...]),
                preferred_element_type=jnp.float32) + m1_sc[...]

    @pl.when(kk == 0)
    def _():
        mx_ref[...] = e

    @pl.when(kk > 0)
    def _():
        mx_ref[...] = jnp.maximum(mx_ref[...], e)

    @pl.when((pl.program_id(0) == 0) & (kk == 0))
    def _():
        st_ref[...] = jnp.zeros_like(st_ref)

    st_ref[...] += jnp.concatenate(
        [jnp.sum(e, axis=0, keepdims=True),
         jnp.sum(e * e, axis=0, keepdims=True),
         jnp.zeros((6, e.shape[1]), jnp.float32)], axis=0)


def _conv(h, nb, wa, wb):
    bn, cin = h.shape
    cout = wa.shape[1]
    tm = 2048
    fl = jax.ShapeDtypeStruct((bn, cout), jnp.float32)
    return pl.pallas_call(
        _conv_body,
        grid=(bn // tm, _NB),
        in_specs=[
            pl.BlockSpec((tm, cin), lambda i, k: (i, 0)),
            pl.BlockSpec((1, tm, cin), lambda i, k: (k, i, 0)),
            pl.BlockSpec((cin, cout), lambda i, k: (0, 0)),
            pl.BlockSpec((cin, cout), lambda i, k: (0, 0)),
        ],
        out_specs=[
            pl.BlockSpec((tm, cout), lambda i, k: (i, 0)),
            pl.BlockSpec((8, cout), lambda i, k: (0, 0)),
        ],
        out_shape=[fl, jax.ShapeDtypeStruct((8, cout), jnp.float32)],
        scratch_shapes=[pltpu.VMEM((tm, cout), jnp.float32)],
    )(h, nb, wa, wb)


def _norm_body(cnt, m_ref, st_ref, gb_ref, o_ref):
    st = st_ref[...]
    mean = st[0:1, :] / cnt
    ex2 = st[1:2, :] / cnt
    var = ex2 - mean * mean
    inv = 1.0 / jnp.sqrt(var + 1e-5)
    pre = (m_ref[...] - mean) * inv * gb_ref[0:1, :] + gb_ref[1:2, :]
    o_ref[...] = jnp.maximum(pre, 0.0)


def _norm(m, st, gb):
    bn, c = m.shape
    cnt = float(_NB * bn)
    return pl.pallas_call(
        functools.partial(_norm_body, cnt),
        grid=(bn // _TM,),
        in_specs=[
            pl.BlockSpec((_TM, c), lambda i: (i, 0)),
            pl.BlockSpec((8, c), lambda i: (0, 0)),
            pl.BlockSpec((8, c), lambda i: (0, 0)),
        ],
        out_specs=pl.BlockSpec((_TM, c), lambda i: (i, 0)),
        out_shape=jax.ShapeDtypeStruct((bn, c), jnp.float32),
    )(m, st, gb)


def _dec_body(h4_ref, h1_ref, co_ref, w4_ref, w1_ref, wc_ref, bb_ref, o_ref):
    acc = jnp.dot(_bf(h4_ref[...]), _bf(w4_ref[...]),
                  preferred_element_type=jnp.float32)
    acc += jnp.dot(_bf(h1_ref[...]), _bf(w1_ref[...]),
                   preferred_element_type=jnp.float32)
    acc += jnp.dot(_bf(co_ref[...]), _bf(wc_ref[...]),
                   preferred_element_type=jnp.float32)
    o_ref[...] = acc + bb_ref[0:1, :]


def _decode(h4, h1, co, w4, w1, wc, bb_p):
    bn = h4.shape[0]
    cout = bb_p.shape[1]
    tm = 2048

    def spec(a):
        return pl.BlockSpec((tm, a.shape[1]), lambda i: (i, 0))

    def wspec(a):
        return pl.BlockSpec(a.shape, lambda i: (0, 0))

    return pl.pallas_call(
        _dec_body,
        grid=(bn // tm,),
        in_specs=[spec(h4), spec(h1), spec(co),
                  wspec(w4), wspec(w1), wspec(wc), wspec(bb_p)],
        out_specs=pl.BlockSpec((tm, cout), lambda i: (i, 0)),
        out_shape=jax.ShapeDtypeStruct((bn, cout), jnp.float32),
    )(h4, h1, co, w4, w1, wc, bb_p)


def _gb(gamma, beta):
    c = gamma.shape[0]
    return jnp.concatenate(
        [gamma[None, :], beta[None, :], jnp.zeros((6, c), jnp.float32)], axis=0)


def _edge_layer(h, idx_kflat, w, gamma, beta):
    cin = w.shape[1] // 2
    wa, wb = w[:, :cin].T, w[:, cin:].T
    if h.shape[1] != cin:
        pad = h.shape[1] - cin
        wa = jnp.concatenate([wa, jnp.zeros((pad, wa.shape[1]), wa.dtype)], axis=0)
        wb = jnp.concatenate([wb, jnp.zeros((pad, wb.shape[1]), wb.dtype)], axis=0)
    nb = _sc_gather(h, idx_kflat)
    mx, st = _conv(h, nb, wa, wb)
    return _norm(mx, st, _gb(gamma, beta))


def kernel(x, coords, W1, g1, b1, W2, g2, b2, W3, g3, b3, W4, g4, b4, Wd, bd):
    bsz, c0, n = x.shape
    bn = bsz * n

    x8 = jnp.concatenate([x, jnp.zeros((bsz, 8 - c0, n), x.dtype)], axis=1)
    idx16 = _knn(x8)
    idx_kflat = jnp.transpose(idx16[:, :, 1:_SEL].reshape(bn, _NB)).reshape(bn * _NB)

    h = jnp.concatenate(
        [jnp.transpose(x, (0, 2, 1)).reshape(bn, c0),
         jnp.zeros((bn, 16 - c0), x.dtype)], axis=1)

    h1 = _edge_layer(h, idx_kflat, W1, g1, b1)
    h2 = _edge_layer(h1, idx_kflat, W2, g2, b2)
    h3 = _edge_layer(h2, idx_kflat, W3, g3, b3)
    h4 = _edge_layer(h3, idx_kflat, W4, g4, b4)

    co = jnp.concatenate(
        [jnp.transpose(coords, (0, 2, 1)).reshape(bn, c0),
         jnp.zeros((bn, 8 - c0), coords.dtype)], axis=1)
    nout = Wd.shape[0]
    c4, c1 = h4.shape[1], h1.shape[1]
    w4 = jnp.zeros((c4, 128), jnp.float32).at[:, :nout].set(Wd[:, :c4].T)
    w1 = jnp.zeros((c1, 128), jnp.float32).at[:, :nout].set(Wd[:, c4:c4 + c1].T)
    wc = jnp.zeros((8, 128), jnp.float32).at[:c0, :nout].set(Wd[:, c4 + c1:].T)
    bb_p = jnp.zeros((8, 128), jnp.float32).at[0, :nout].set(bd)

    out = _decode(h4, h1, co, w4, w1, wc, bb_p)[:, :nout]
    return jnp.transpose(out.reshape(bsz, n, nout), (0, 2, 1))


# speedup vs baseline: 13.2683x; 1.0543x over previous
import functools

import jax
import jax.numpy as jnp
from jax import lax
from jax.experimental import pallas as pl
from jax.experimental.pallas import tpu as pltpu
from jax.experimental.pallas import tpu_sc as plsc

_NB = 9
_SEL = _NB + 1
_TN = 1024
_TM = 512
_NC, _NS = 2, 16
_G = 64


def _bf(x):
    return x.astype(jnp.bfloat16)


def _knn_body(n, xa_ref, xt_ref, o_ref):
    b = pl.program_id(0)
    a = xa_ref[0]
    rt = xt_ref[0]
    inner = lax.dot_general(_bf(rt), _bf(a), (((0,), (0,)), ((), ())),
                            preferred_element_type=jnp.float32)
    sq = jnp.sum(a * a, axis=0, keepdims=True)
    sqr = jnp.sum(rt * rt, axis=0)[:, None]
    dist = (sqr + sq) - 2.0 * inner

    iota = lax.broadcasted_iota(jnp.int32, (_TN, n), 1)
    coli = lax.broadcasted_iota(jnp.int32, (_TN, 16), 1)
    cols = jnp.zeros((_TN, 16), jnp.int32)
    for t in range(_SEL):
        m = jnp.min(dist, axis=1, keepdims=True)
        am = jnp.min(jnp.where(dist == m, iota, n), axis=1, keepdims=True)
        cols = jnp.where(coli == t, am + b * n, cols)
        dist = jnp.where(iota == am, jnp.float32(jnp.inf), dist)
    o_ref[0] = cols


def _knn(x8):
    bsz, _, n = x8.shape
    return pl.pallas_call(
        functools.partial(_knn_body, n),
        grid=(bsz, n // _TN),
        in_specs=[
            pl.BlockSpec((1, 8, n), lambda b, j: (b, 0, 0)),
            pl.BlockSpec((1, 8, _TN), lambda b, j: (b, 0, j)),
        ],
        out_specs=pl.BlockSpec((1, _TN, 16), lambda b, j: (b, j, 0)),
        out_shape=jax.ShapeDtypeStruct((bsz, n, 16), jnp.int32),
    )(x8, x8)


def _sc_gather(h, idx_kflat):
    bn, c = h.shape
    nw = _NC * _NS
    ppw = bn // nw
    nchunk = ppw // _G
    mesh = plsc.VectorSubcoreMesh(core_axis_name="c", subcore_axis_name="s",
                                  num_cores=_NC, num_subcores=_NS)

    gg = 128
    nch = ppw // gg
    t_total = _NB * nch

    @functools.partial(
        pl.kernel,
        out_type=jax.ShapeDtypeStruct((_NB, bn, c), jnp.float32),
        mesh=mesh,
        compiler_params=pltpu.CompilerParams(use_tc_tiling_on_sc=False),
        scratch_types=[
            pltpu.VMEM((_NB * ppw,), jnp.int32),
            pltpu.VMEM((gg, c), jnp.float32),
            pltpu.VMEM((gg, c), jnp.float32),
            pltpu.SemaphoreType.DMA,
            pltpu.SemaphoreType.DMA,
            pltpu.SemaphoreType.DMA,
            pltpu.SemaphoreType.DMA,
        ])
    def k(h_hbm, idx_hbm, nb_hbm, idx_all, rows0, rows1, g0, g1, s0, s1):
        wid = lax.axis_index("s") * _NC + lax.axis_index("c")
        base = wid * ppw
        rows = (rows0, rows1)
        gsem = (g0, g1)
        ssem = (s0, s1)

        for kk in range(_NB):
            pltpu.sync_copy(idx_hbm.at[pl.ds(kk * bn + base, ppw)],
                            idx_all.at[pl.ds(kk * ppw, ppw)])

        gath = [None, None]
        stor = [None, None]
        for t in range(t_total):
            b = t % 2
            if t >= 2:
                stor[b].wait()
            kk, cc = divmod(t, nch)
            gath[b] = pltpu.async_copy(
                h_hbm.at[idx_all.at[pl.ds(kk * ppw + cc * gg, gg)]],
                rows[b], gsem[b])
            if t >= 1:
                bp = (t - 1) % 2
                gath[bp].wait()
                kp, cp = divmod(t - 1, nch)
                stor[bp] = pltpu.async_copy(
                    rows[bp], nb_hbm.at[kp, pl.ds(base + cp * gg, gg)],
                    ssem[bp])
        bl = (t_total - 1) % 2
        gath[bl].wait()
        kp, cp = divmod(t_total - 1, nch)
        stor[bl] = pltpu.async_copy(
            rows[bl], nb_hbm.at[kp, pl.ds(base + cp * gg, gg)], ssem[bl])
        stor[1 - bl].wait()
        stor[bl].wait()

    return k(h, idx_kflat)


def _conv_body(h_ref, nb_ref, wa_ref, wb_ref, mx_ref, st_ref, m1_sc):
    kk = pl.program_id(1)
    h = h_ref[...]

    @pl.when(kk == 0)
    def _():
        m1_sc[...] = jnp.dot(_bf(h), _bf(wa_ref[...]),
                             preferred_element_type=jnp.float32)

    d = _bf(nb_ref[0] - h)
    e = jnp.dot(d, _bf(wb_ref[...]),
                preferred_element_type=jnp.float32) + m1_sc[...]

    @pl.when(kk == 0)
    def _():
        mx_ref[...] = e

    @pl.when(kk > 0)
    def _():
        mx_ref[...] = jnp.maximum(mx_ref[...], e)

    @pl.when((pl.program_id(0) == 0) & (kk == 0))
    def _():
        st_ref[...] = jnp.zeros_like(st_ref)

    st_ref[...] += jnp.concatenate(
        [jnp.sum(e, axis=0, keepdims=True),
         jnp.sum(e * e, axis=0, keepdims=True),
         jnp.zeros((6, e.shape[1]), jnp.float32)], axis=0)


def _conv(h, nb, wa, wb):
    bn, cin = h.shape
    cout = wa.shape[1]
    tm = 4096
    fl = jax.ShapeDtypeStruct((bn, cout), jnp.float32)
    return pl.pallas_call(
        _conv_body,
        grid=(bn // tm, _NB),
        in_specs=[
            pl.BlockSpec((tm, cin), lambda i, k: (i, 0)),
            pl.BlockSpec((1, tm, cin), lambda i, k: (k, i, 0)),
            pl.BlockSpec((cin, cout), lambda i, k: (0, 0)),
            pl.BlockSpec((cin, cout), lambda i, k: (0, 0)),
        ],
        out_specs=[
            pl.BlockSpec((tm, cout), lambda i, k: (i, 0)),
            pl.BlockSpec((8, cout), lambda i, k: (0, 0)),
        ],
        out_shape=[fl, jax.ShapeDtypeStruct((8, cout), jnp.float32)],
        scratch_shapes=[pltpu.VMEM((tm, cout), jnp.float32)],
    )(h, nb, wa, wb)


def _norm_body(cnt, m_ref, st_ref, gb_ref, o_ref):
    st = st_ref[...]
    mean = st[0:1, :] / cnt
    ex2 = st[1:2, :] / cnt
    var = ex2 - mean * mean
    inv = 1.0 / jnp.sqrt(var + 1e-5)
    pre = (m_ref[...] - mean) * inv * gb_ref[0:1, :] + gb_ref[1:2, :]
    o_ref[...] = jnp.maximum(pre, 0.0)


def _norm(m, st, gb):
    bn, c = m.shape
    cnt = float(_NB * bn)
    return pl.pallas_call(
        functools.partial(_norm_body, cnt),
        grid=(bn // _TM,),
        in_specs=[
            pl.BlockSpec((_TM, c), lambda i: (i, 0)),
            pl.BlockSpec((8, c), lambda i: (0, 0)),
            pl.BlockSpec((8, c), lambda i: (0, 0)),
        ],
        out_specs=pl.BlockSpec((_TM, c), lambda i: (i, 0)),
        out_shape=jax.ShapeDtypeStruct((bn, c), jnp.float32),
    )(m, st, gb)


def _dec_body(h4_ref, h1_ref, co_ref, w4_ref, w1_ref, wc_ref, bb_ref, o_ref):
    acc = jnp.dot(_bf(h4_ref[...]), _bf(w4_ref[...]),
                  preferred_element_type=jnp.float32)
    acc += jnp.dot(_bf(h1_ref[...]), _bf(w1_ref[...]),
                   preferred_element_type=jnp.float32)
    acc += jnp.dot(_bf(co_ref[...]), _bf(wc_ref[...]),
                   preferred_element_type=jnp.float32)
    o_ref[...] = acc + bb_ref[0:1, :]


def _decode(h4, h1, co, w4, w1, wc, bb_p):
    bn = h4.shape[0]
    cout = bb_p.shape[1]
    tm = 2048

    def spec(a):
        return pl.BlockSpec((tm, a.shape[1]), lambda i: (i, 0))

    def wspec(a):
        return pl.BlockSpec(a.shape, lambda i: (0, 0))

    return pl.pallas_call(
        _dec_body,
        grid=(bn // tm,),
        in_specs=[spec(h4), spec(h1), spec(co),
                  wspec(w4), wspec(w1), wspec(wc), wspec(bb_p)],
        out_specs=pl.BlockSpec((tm, cout), lambda i: (i, 0)),
        out_shape=jax.ShapeDtypeStruct((bn, cout), jnp.float32),
    )(h4, h1, co, w4, w1, wc, bb_p)


def _gb(gamma, beta):
    c = gamma.shape[0]
    return jnp.concatenate(
        [gamma[None, :], beta[None, :], jnp.zeros((6, c), jnp.float32)], axis=0)


def _edge_layer(h, idx_kflat, w, gamma, beta):
    cin = w.shape[1] // 2
    wa, wb = w[:, :cin].T, w[:, cin:].T
    if h.shape[1] != cin:
        pad = h.shape[1] - cin
        wa = jnp.concatenate([wa, jnp.zeros((pad, wa.shape[1]), wa.dtype)], axis=0)
        wb = jnp.concatenate([wb, jnp.zeros((pad, wb.shape[1]), wb.dtype)], axis=0)
    nb = _sc_gather(h, idx_kflat)
    mx, st = _conv(h, nb, wa, wb)
    return _norm(mx, st, _gb(gamma, beta))


def kernel(x, coords, W1, g1, b1, W2, g2, b2, W3, g3, b3, W4, g4, b4, Wd, bd):
    bsz, c0, n = x.shape
    bn = bsz * n

    x8 = jnp.concatenate([x, jnp.zeros((bsz, 8 - c0, n), x.dtype)], axis=1)
    idx16 = _knn(x8)
    idx_kflat = jnp.transpose(idx16[:, :, 1:_SEL].reshape(bn, _NB)).reshape(bn * _NB)

    h = jnp.concatenate(
        [jnp.transpose(x, (0, 2, 1)).reshape(bn, c0),
         jnp.zeros((bn, 16 - c0), x.dtype)], axis=1)

    h1 = _edge_layer(h, idx_kflat, W1, g1, b1)
    h2 = _edge_layer(h1, idx_kflat, W2, g2, b2)
    h3 = _edge_layer(h2, idx_kflat, W3, g3, b3)
    h4 = _edge_layer(h3, idx_kflat, W4, g4, b4)

    co = jnp.concatenate(
        [jnp.transpose(coords, (0, 2, 1)).reshape(bn, c0),
         jnp.zeros((bn, 8 - c0), coords.dtype)], axis=1)
    nout = Wd.shape[0]
    c4, c1 = h4.shape[1], h1.shape[1]
    w4 = jnp.zeros((c4, 128), jnp.float32).at[:, :nout].set(Wd[:, :c4].T)
    w1 = jnp.zeros((c1, 128), jnp.float32).at[:, :nout].set(Wd[:, c4:c4 + c1].T)
    wc = jnp.zeros((8, 128), jnp.float32).at[:c0, :nout].set(Wd[:, c4 + c1:].T)
    bb_p = jnp.zeros((8, 128), jnp.float32).at[0, :nout].set(bd)

    out = _decode(h4, h1, co, w4, w1, wc, bb_p)[:, :nout]
    return jnp.transpose(out.reshape(bsz, n, nout), (0, 2, 1))
